# baseline (speedup 1.0000x reference)
import functools

import jax
import jax.numpy as jnp
from jax import lax
from jax.experimental import pallas as pl
from jax.experimental.pallas import tpu as pltpu
from jax.experimental.pallas import tpu_sc as plsc

B = 16384
N_NUM = 13
N_CAT = 26
VOCAB = 100001
EMB = 50

_INFO = plsc.get_sparse_core_info()
NC = _INFO.num_cores
NS = _INFO.num_subcores
NW = NC * NS

TOTAL_ROWS = B * N_CAT
IDX_W = 128
ROWS_PER_W = TOTAL_ROWS // NW
CHUNK = 1024
G = CHUNK // IDX_W
N_CHUNKS = ROWS_PER_W // CHUNK
EMB_PAD = 56


def _emb_gather(idx2d, padded_tables):
    mesh = plsc.VectorSubcoreMesh(core_axis_name="c", subcore_axis_name="s")

    @functools.partial(
        pl.kernel,
        mesh=mesh,
        out_type=jax.ShapeDtypeStruct((TOTAL_ROWS, EMB_PAD), jnp.float32),
        scratch_types=[
            pltpu.VMEM((G, IDX_W), jnp.int32),
            pltpu.VMEM((CHUNK, EMB_PAD), jnp.float32),
            pltpu.SemaphoreType.DMA,
        ],
        compiler_params=pltpu.CompilerParams(use_tc_tiling_on_sc=False),
    )
    def k(idx_hbm, table_hbm, out_hbm, idx_v, rows_v, sem):
        wid = lax.axis_index("s") * NC + lax.axis_index("c")
        idx_row0 = wid * (ROWS_PER_W // IDX_W)
        out_row0 = wid * ROWS_PER_W

        def body(ci, _):
            pltpu.sync_copy(idx_hbm.at[pl.ds(idx_row0 + ci * G, G)], idx_v)
            copies = [
                pltpu.async_copy(
                    table_hbm.at[idx_v.at[j]],
                    rows_v.at[pl.ds(j * IDX_W, IDX_W)],
                    sem,
                )
                for j in range(G)
            ]
            for c in copies:
                c.wait()
            pltpu.sync_copy(
                rows_v, out_hbm.at[pl.ds(out_row0 + ci * CHUNK, CHUNK)]
            )
            return 0

        lax.fori_loop(0, N_CHUNKS, body, 0)

    return k(idx2d, padded_tables)


def kernel(x_num, x_cat, tables):
    flat_tables = tables.reshape(N_CAT * VOCAB, EMB)
    padded_tables = jnp.pad(flat_tables, ((0, 0), (0, EMB_PAD - EMB)))
    idx = x_cat + jnp.arange(N_CAT, dtype=jnp.int32) * VOCAB
    idx2d = idx.reshape(TOTAL_ROWS // IDX_W, IDX_W)
    emb = _emb_gather(idx2d, padded_tables)
    emb = emb.reshape(B, N_CAT, EMB_PAD)[:, :, :EMB].reshape(B, N_CAT * EMB)
    return jnp.concatenate([x_num, emb], axis=-1)


# speedup vs baseline: 1.3452x; 1.3452x over previous
import functools

import jax
import jax.numpy as jnp
from jax import lax
from jax.experimental import pallas as pl
from jax.experimental.pallas import tpu as pltpu
from jax.experimental.pallas import tpu_sc as plsc

B = 16384
N_NUM = 13
N_CAT = 26
VOCAB = 100001
EMB = 50
EMB_PAD = 128

_INFO = plsc.get_sparse_core_info()
NC = _INFO.num_cores
NS = _INFO.num_subcores
NW = NC * NS

TOTAL_ROWS = B * N_CAT
IDX_W = 128
ROWS_PER_W = TOTAL_ROWS // NW
CHUNK = 512
G = CHUNK // IDX_W
N_CHUNKS = ROWS_PER_W // CHUNK


def _emb_gather(idx2d, padded_tables):
    mesh = plsc.VectorSubcoreMesh(core_axis_name="c", subcore_axis_name="s")

    @functools.partial(
        pl.kernel,
        mesh=mesh,
        out_type=jax.ShapeDtypeStruct((TOTAL_ROWS, EMB_PAD), jnp.float32),
        scratch_types=[
            pltpu.VMEM((G, IDX_W), jnp.int32),
            pltpu.VMEM((CHUNK, EMB_PAD), jnp.float32),
            pltpu.SemaphoreType.DMA,
        ],
        compiler_params=pltpu.CompilerParams(use_tc_tiling_on_sc=True),
    )
    def k(idx_hbm, table_hbm, out_hbm, idx_v, rows_v, sem):
        wid = lax.axis_index("s") * NC + lax.axis_index("c")
        idx_row0 = wid * (ROWS_PER_W // IDX_W)
        out_row0 = wid * ROWS_PER_W

        def body(ci, _):
            pltpu.sync_copy(idx_hbm.at[pl.ds(idx_row0 + ci * G, G)], idx_v)
            copies = [
                pltpu.async_copy(
                    table_hbm.at[idx_v.at[j]],
                    rows_v.at[pl.ds(j * IDX_W, IDX_W)],
                    sem,
                )
                for j in range(G)
            ]
            for c in copies:
                c.wait()
            pltpu.sync_copy(
                rows_v, out_hbm.at[pl.ds(out_row0 + ci * CHUNK, CHUNK)]
            )
            return 0

        lax.fori_loop(0, N_CHUNKS, body, 0)

    return k(idx2d, padded_tables)


def kernel(x_num, x_cat, tables):
    flat_tables = tables.reshape(N_CAT * VOCAB, EMB)
    padded_tables = jnp.pad(flat_tables, ((0, 0), (0, EMB_PAD - EMB)))
    idx = x_cat + jnp.arange(N_CAT, dtype=jnp.int32) * VOCAB
    idx2d = idx.reshape(TOTAL_ROWS // IDX_W, IDX_W)
    emb = _emb_gather(idx2d, padded_tables)
    emb = emb.reshape(B, N_CAT, EMB_PAD)[:, :, :EMB].reshape(B, N_CAT * EMB)
    return jnp.concatenate([x_num, emb], axis=-1)
